# Initial kernel scaffold; baseline (speedup 1.0000x reference)
import jax
import jax.numpy as jnp
from jax.experimental import pallas as pl


def kernel(x_1, incidence_1, W0, b0, W1, b1, W_out, b_out):
    raise NotImplementedError("write your pallas kernel here")


# speedup vs baseline: 1.1037x; 1.1037x over previous
import functools

import jax
import jax.numpy as jnp
from jax.experimental import pallas as pl
from jax.experimental.pallas import tpu as pltpu


def _unisage_body(x_ref, a_ref, w_ref, b_ref, wout_ref, bout_ref,
                  out_ref, x_state, m_acc, deg, *, n_layers, k_blocks, valid_last):
    l = pl.program_id(0)
    k = pl.program_id(1)

    @pl.when((l == 0) & (k == 0))
    def _():
        x_state[...] = x_ref[...]

    @pl.when(k == 0)
    def _():
        x_state[...] = jax.lax.dot_general(
            x_state[...], w_ref[0],
            (((1,), (1,)), ((), ())),
            preferred_element_type=jnp.float32,
        ) + b_ref[0]

    def step(w):
        m01 = jax.lax.dot_general(
            a_ref[:, :w], x_state[...], (((0,), (0,)), ((), ())),
            preferred_element_type=jnp.float32)
        contrib = jax.lax.dot_general(
            a_ref[:, :w], m01, (((1,), (0,)), ((), ())),
            preferred_element_type=jnp.float32)

        @pl.when(k == 0)
        def _():
            m_acc[...] = contrib

        @pl.when(k > 0)
        def _():
            m_acc[...] += contrib

        @pl.when(l == 0)
        def _():
            dblk = jnp.sum(a_ref[:, :w], axis=1, keepdims=True)

            @pl.when(k == 0)
            def _():
                deg[...] = dblk

            @pl.when(k > 0)
            def _():
                deg[...] += dblk

    full_bk = a_ref.shape[1]
    if valid_last == full_bk:
        step(full_bk)
    else:
        @pl.when(k < k_blocks - 1)
        def _():
            step(full_bk)

        @pl.when(k == k_blocks - 1)
        def _():
            step(valid_last)

    @pl.when(k == k_blocks - 1)
    def _():
        x_state[...] = x_state[...] + m_acc[...] / deg[...]

    @pl.when((l == n_layers - 1) & (k == k_blocks - 1))
    def _():
        pooled = jnp.max(x_state[...], axis=0, keepdims=True)
        logit = jnp.sum(pooled * wout_ref[...], axis=1, keepdims=True)
        out_ref[...] = jax.nn.sigmoid(logit + bout_ref[...])


@jax.jit
def kernel(x_1, incidence_1, W0, b0, W1, b1, W_out, b_out):
    n, d = x_1.shape
    n_layers = 2
    bk = min(256, n)
    k_blocks = -(-n // bk)
    valid_last = n - (k_blocks - 1) * bk

    ws = jnp.stack([W0, W1])
    bs = jnp.stack([b0, b1]).reshape(n_layers, 1, d)
    bout = b_out.reshape(1, 1)

    grid = (n_layers, k_blocks)
    out = pl.pallas_call(
        functools.partial(_unisage_body, n_layers=n_layers,
                          k_blocks=k_blocks, valid_last=valid_last),
        grid=grid,
        in_specs=[
            pl.BlockSpec((n, d), lambda l, k: (0, 0)),
            pl.BlockSpec((n, bk), lambda l, k: (0, k)),
            pl.BlockSpec((1, d, d), lambda l, k: (l, 0, 0)),
            pl.BlockSpec((1, 1, d), lambda l, k: (l, 0, 0)),
            pl.BlockSpec((1, d), lambda l, k: (0, 0)),
            pl.BlockSpec((1, 1), lambda l, k: (0, 0)),
        ],
        out_specs=pl.BlockSpec((1, 1), lambda l, k: (0, 0)),
        out_shape=jax.ShapeDtypeStruct((1, 1), jnp.float32),
        scratch_shapes=[
            pltpu.VMEM((n, d), jnp.float32),
            pltpu.VMEM((n, d), jnp.float32),
            pltpu.VMEM((n, 1), jnp.float32),
        ],
        compiler_params=pltpu.CompilerParams(
            dimension_semantics=("arbitrary", "arbitrary"),
            vmem_limit_bytes=60 * 1024 * 1024,
        ),
    )(x_1, incidence_1, ws, bs, W_out, bout)
    return out.reshape(1)


# speedup vs baseline: 1.7251x; 1.5630x over previous
import functools

import jax
import jax.numpy as jnp
from jax.experimental import pallas as pl
from jax.experimental.pallas import tpu as pltpu


def _unisage_body(x_ref, a_ref, w_ref, b_ref, wout_ref, bout_ref,
                  out_ref, x_state, x_bf, a_bf, m_acc, dacc, deg,
                  *, n_layers, k_blocks, valid_last):
    l = pl.program_id(0)
    k = pl.program_id(1)
    d = x_ref.shape[1]

    @pl.when((l == 0) & (k == 0))
    def _():
        x_state[...] = x_ref[...]
        dacc[...] = jnp.zeros(dacc.shape, dacc.dtype)

    @pl.when(k == 0)
    def _():
        xl = jax.lax.dot_general(
            x_state[...], w_ref[0],
            (((1,), (1,)), ((), ())),
            preferred_element_type=jnp.float32,
        ) + b_ref[0]
        x_state[...] = xl
        x_bf[...] = xl.astype(jnp.bfloat16)

    def step(w):
        a_bf[:, :w] = a_ref[:, :w].astype(jnp.bfloat16)
        m01 = jax.lax.dot_general(
            a_bf[:, :w], x_bf[...], (((0,), (0,)), ((), ())),
            preferred_element_type=jnp.float32)
        contrib = jax.lax.dot_general(
            a_bf[:, :w], m01.astype(jnp.bfloat16), (((1,), (0,)), ((), ())),
            preferred_element_type=jnp.float32)

        @pl.when(k == 0)
        def _():
            m_acc[...] = contrib

        @pl.when(k > 0)
        def _():
            m_acc[...] += contrib

        @pl.when(l == 0)
        def _():
            for c in range(0, w, d):
                e = min(c + d, w)
                dacc[:, :e - c] += a_ref[:, c:e]

    full_bk = a_ref.shape[1]
    if valid_last == full_bk:
        step(full_bk)
    else:
        @pl.when(k < k_blocks - 1)
        def _():
            step(full_bk)

        @pl.when(k == k_blocks - 1)
        def _():
            step(valid_last)

    @pl.when((l == 0) & (k == k_blocks - 1))
    def _():
        deg[...] = jnp.sum(dacc[...], axis=1, keepdims=True)

    @pl.when(k == k_blocks - 1)
    def _():
        x_state[...] = x_state[...] + m_acc[...] / deg[...]

    @pl.when((l == n_layers - 1) & (k == k_blocks - 1))
    def _():
        pooled = jnp.max(x_state[...], axis=0, keepdims=True)
        logit = jnp.sum(pooled * wout_ref[...], axis=1, keepdims=True)
        out_ref[...] = jax.nn.sigmoid(logit + bout_ref[...])


@jax.jit
def kernel(x_1, incidence_1, W0, b0, W1, b1, W_out, b_out):
    n, d = x_1.shape
    n_layers = 2
    bk = min(256, n)
    k_blocks = -(-n // bk)
    valid_last = n - (k_blocks - 1) * bk

    ws = jnp.stack([W0, W1])
    bs = jnp.stack([b0, b1]).reshape(n_layers, 1, d)
    bout = b_out.reshape(1, 1)

    grid = (n_layers, k_blocks)
    out = pl.pallas_call(
        functools.partial(_unisage_body, n_layers=n_layers,
                          k_blocks=k_blocks, valid_last=valid_last),
        grid=grid,
        in_specs=[
            pl.BlockSpec((n, d), lambda l, k: (0, 0)),
            pl.BlockSpec((n, bk), lambda l, k: (0, k)),
            pl.BlockSpec((1, d, d), lambda l, k: (l, 0, 0)),
            pl.BlockSpec((1, 1, d), lambda l, k: (l, 0, 0)),
            pl.BlockSpec((1, d), lambda l, k: (0, 0)),
            pl.BlockSpec((1, 1), lambda l, k: (0, 0)),
        ],
        out_specs=pl.BlockSpec((1, 1), lambda l, k: (0, 0)),
        out_shape=jax.ShapeDtypeStruct((1, 1), jnp.float32),
        scratch_shapes=[
            pltpu.VMEM((n, d), jnp.float32),
            pltpu.VMEM((n, d), jnp.bfloat16),
            pltpu.VMEM((n, bk), jnp.bfloat16),
            pltpu.VMEM((n, d), jnp.float32),
            pltpu.VMEM((n, d), jnp.float32),
            pltpu.VMEM((n, 1), jnp.float32),
        ],
        compiler_params=pltpu.CompilerParams(
            dimension_semantics=("arbitrary", "arbitrary"),
            vmem_limit_bytes=60 * 1024 * 1024,
        ),
    )(x_1, incidence_1, ws, bs, W_out, bout)
    return out.reshape(1)
